# Initial kernel scaffold; baseline (speedup 1.0000x reference)
import jax
import jax.numpy as jnp
from jax.experimental import pallas as pl


def kernel(pred_coords, true_coords, pred_atoms, atoms_target, pred_charges, charges_target, pred_bonds, bonds_target, batch, bond_aggregation_index, weights):
    raise NotImplementedError("write your pallas kernel here")


# speedup vs baseline: 8.2162x; 8.2162x over previous
import functools

import jax
import jax.numpy as jnp
from jax import lax
from jax.experimental import pallas as pl
from jax.experimental.pallas import tpu as pltpu
from jax.experimental.pallas import tpu_sc as plsc

LANES = 128
NTILES = 32
SCL = 16


def _ceil_to(x, m):
    return (x + m - 1) // m * m


def _ce_kernel(x_ref, t_ref, o_ref):
    x = x_ref[0]
    c = x.shape[0]
    m = jnp.max(x, axis=0, keepdims=True)
    e = jnp.exp(x - m)
    s = jnp.sum(e, axis=0, keepdims=True)
    lse = jnp.log(s) + m
    lbl = t_ref[0]
    iota = lax.broadcasted_iota(jnp.int32, (c, x.shape[1]), 0)
    picked = jnp.sum(jnp.where(iota == lbl, x, 0.0), axis=0, keepdims=True)
    o_ref[0] = lse - picked


def _atom_kernel(a_ref, t_ref, pc_ref, tc_ref, ce_ref, mse_ref):
    _ce_kernel(a_ref, t_ref, ce_ref)
    d = pc_ref[0] - tc_ref[0]
    mse_ref[0] = jnp.sum(d * d, axis=0, keepdims=True) * (1.0 / 3.0)


def _sc_scatter(nt_e, nt_a, na, bp,
                ce2d, idx2d, mse2d, cea2d, bat2d, zeros_hbm,
                s_out, c_out, n_out, ms_out, cs_out,
                s_sp, c_sp, n_sp, ms_sp, cs_sp,
                cev, idxv, msev, ceav, batv, ones_v):
    ci = lax.axis_index("c")
    si = lax.axis_index("s")
    wid = ci * 16 + si

    @pl.when(si == 0)
    def _init():
        pltpu.sync_copy(zeros_hbm.at[pl.ds(0, na)], s_sp)
        pltpu.sync_copy(zeros_hbm.at[pl.ds(0, na)], c_sp)
        pltpu.sync_copy(zeros_hbm.at[pl.ds(0, bp)], n_sp)
        pltpu.sync_copy(zeros_hbm.at[pl.ds(0, bp)], ms_sp)
        pltpu.sync_copy(zeros_hbm.at[pl.ds(0, bp)], cs_sp)

    for k in range(LANES // SCL):
        ones_v[pl.ds(k * SCL, SCL)] = jnp.full((SCL,), 1.0, jnp.float32)

    pltpu.sync_copy(ce2d.at[pl.ds(wid * nt_e, nt_e)], cev)
    pltpu.sync_copy(idx2d.at[pl.ds(wid * nt_e, nt_e)], idxv)
    pltpu.sync_copy(mse2d.at[pl.ds(wid * nt_a, nt_a)], msev)
    pltpu.sync_copy(cea2d.at[pl.ds(wid * nt_a, nt_a)], ceav)
    pltpu.sync_copy(bat2d.at[pl.ds(wid * nt_a, nt_a)], batv)

    plsc.subcore_barrier()

    for r in range(nt_e):
        pltpu.sync_copy(cev.at[r], s_sp.at[idxv.at[r]], add=True)
        pltpu.sync_copy(ones_v, c_sp.at[idxv.at[r]], add=True)
    for r in range(nt_a):
        pltpu.sync_copy(msev.at[r], ms_sp.at[batv.at[r]], add=True)
        pltpu.sync_copy(ceav.at[r], cs_sp.at[batv.at[r]], add=True)
        pltpu.sync_copy(ones_v, n_sp.at[batv.at[r]], add=True)

    plsc.subcore_barrier()

    @pl.when(si == 0)
    def _emit():
        pltpu.sync_copy(s_sp, s_out.at[pl.ds(ci * na, na)])
        pltpu.sync_copy(c_sp, c_out.at[pl.ds(ci * na, na)])
        pltpu.sync_copy(n_sp, n_out.at[pl.ds(ci * bp, bp)])
        pltpu.sync_copy(ms_sp, ms_out.at[pl.ds(ci * bp, bp)])
        pltpu.sync_copy(cs_sp, cs_out.at[pl.ds(ci * bp, bp)])


def _sc_final(rows, na, bp,
              s2, c2, n2, ms2, cs2, w_pad, bat2d, zeros_hbm,
              bonds_out, coords_out, atoms_out, charges_out,
              tb_sp,
              s0v, s1v, c0v, c1v, batv, tv, wv, n0v, n1v, msv, csv,
              wdivn_v, accv):
    ci = lax.axis_index("c")
    si = lax.axis_index("s")
    wid = ci * 16 + si
    nb_v = bp // SCL
    ch = rows * LANES

    @pl.when(si == 0)
    def _init():
        pltpu.sync_copy(zeros_hbm.at[pl.ds(0, bp)], tb_sp)

    pltpu.sync_copy(w_pad, wv)
    pltpu.sync_copy(n2.at[pl.ds(0, bp)], n0v)
    pltpu.sync_copy(n2.at[pl.ds(bp, bp)], n1v)
    for k in range(nb_v):
        d = pl.ds(k * SCL, SCL)
        nv = n0v[d] + n1v[d]
        wdivn_v[d] = wv[d] / jnp.maximum(nv, 1.0)

    base = wid * ch
    pltpu.sync_copy(s2.at[pl.ds(base, ch)], s0v)
    pltpu.sync_copy(s2.at[pl.ds(na + base, ch)], s1v)
    pltpu.sync_copy(c2.at[pl.ds(base, ch)], c0v)
    pltpu.sync_copy(c2.at[pl.ds(na + base, ch)], c1v)
    pltpu.sync_copy(bat2d.at[pl.ds(wid * rows, rows)], batv)

    for r in range(rows):
        for j in range(LANES // SCL):
            d = pl.ds(r * LANES + j * SCL, SCL)
            sv = s0v[d] + s1v[d]
            cv = c0v[d] + c1v[d]
            tv[r, pl.ds(j * SCL, SCL)] = 0.5 * sv / jnp.maximum(cv, 1.0)

    plsc.subcore_barrier()
    for r in range(rows):
        pltpu.sync_copy(tv.at[r], tb_sp.at[batv.at[r]], add=True)
    plsc.subcore_barrier()

    @pl.when(si == 0)
    def _emit_bonds():
        pltpu.sync_copy(tb_sp, msv)
        bacc = jnp.zeros((SCL,), jnp.float32)
        for k in range(nb_v):
            d = pl.ds(k * SCL, SCL)
            bacc = bacc + msv[d] * wdivn_v[d]
        accv[...] = bacc
        pltpu.sync_copy(accv, bonds_out.at[pl.ds(ci * SCL, SCL)])

    @pl.when((si == 0) & (ci == 0))
    def _scalars():
        pltpu.sync_copy(ms2.at[pl.ds(0, bp)], msv)
        pltpu.sync_copy(cs2.at[pl.ds(0, bp)], csv)
        cacc = jnp.zeros((SCL,), jnp.float32)
        aacc = jnp.zeros((SCL,), jnp.float32)
        wacc = jnp.zeros((SCL,), jnp.float32)
        for k in range(nb_v):
            d = pl.ds(k * SCL, SCL)
            cacc = cacc + msv[d] * wdivn_v[d]
            aacc = aacc + csv[d] * wdivn_v[d]
            wacc = wacc + wv[d]
        pltpu.sync_copy(ms2.at[pl.ds(bp, bp)], msv)
        pltpu.sync_copy(cs2.at[pl.ds(bp, bp)], csv)
        for k in range(nb_v):
            d = pl.ds(k * SCL, SCL)
            cacc = cacc + msv[d] * wdivn_v[d]
            aacc = aacc + csv[d] * wdivn_v[d]
        accv[...] = cacc
        pltpu.sync_copy(accv, coords_out)
        accv[...] = aacc
        pltpu.sync_copy(accv, atoms_out)
        accv[...] = wacc
        pltpu.sync_copy(accv, charges_out)


def kernel(pred_coords, true_coords, pred_atoms, atoms_target, pred_charges,
           charges_target, pred_bonds, bonds_target, batch,
           bond_aggregation_index, weights):
    n = pred_coords.shape[0]
    e = pred_bonds.shape[0]
    b = weights.shape[0]
    a_cls = pred_atoms.shape[1]
    bond_cls = pred_bonds.shape[1]

    nt_e = _ceil_to(-(-e // (NTILES * LANES)), 8)
    ep = nt_e * NTILES * LANES
    nt_a = _ceil_to(-(-n // (NTILES * LANES)), 8)
    na = nt_a * NTILES * LANES
    bp = _ceil_to(b + 1, LANES)

    f32 = jnp.float32
    i32 = jnp.int32

    KE = NTILES * LANES
    ge = ep // KE
    pb = jnp.pad(pred_bonds, ((0, ep - e), (0, 0)))
    pb3 = jnp.transpose(pb.reshape(ge, KE, bond_cls), (0, 2, 1))
    bt3 = jnp.pad(bonds_target, (0, ep - e)).reshape(ge, 1, KE)

    ga = na // KE
    pa = jnp.pad(pred_atoms, ((0, na - n), (0, 0)))
    pa3 = jnp.transpose(pa.reshape(ga, KE, a_cls), (0, 2, 1))
    at3 = jnp.pad(atoms_target, (0, na - n)).reshape(ga, 1, KE)
    pc3 = jnp.transpose(jnp.pad(pred_coords, ((0, na - n), (0, 0))).reshape(ga, KE, 3), (0, 2, 1))
    tc3 = jnp.transpose(jnp.pad(true_coords, ((0, na - n), (0, 0))).reshape(ga, KE, 3), (0, 2, 1))

    ce_b3 = pl.pallas_call(
        _ce_kernel,
        grid=(ge,),
        in_specs=[
            pl.BlockSpec((1, bond_cls, KE), lambda i: (i, 0, 0)),
            pl.BlockSpec((1, 1, KE), lambda i: (i, 0, 0)),
        ],
        out_specs=pl.BlockSpec((1, 1, KE), lambda i: (i, 0, 0)),
        out_shape=jax.ShapeDtypeStruct((ge, 1, KE), f32),
    )(pb3, bt3)

    ce_a3, mse3 = pl.pallas_call(
        _atom_kernel,
        grid=(ga,),
        in_specs=[
            pl.BlockSpec((1, a_cls, KE), lambda i: (i, 0, 0)),
            pl.BlockSpec((1, 1, KE), lambda i: (i, 0, 0)),
            pl.BlockSpec((1, 3, KE), lambda i: (i, 0, 0)),
            pl.BlockSpec((1, 3, KE), lambda i: (i, 0, 0)),
        ],
        out_specs=[
            pl.BlockSpec((1, 1, KE), lambda i: (i, 0, 0)),
            pl.BlockSpec((1, 1, KE), lambda i: (i, 0, 0)),
        ],
        out_shape=[
            jax.ShapeDtypeStruct((ga, 1, KE), f32),
            jax.ShapeDtypeStruct((ga, 1, KE), f32),
        ],
    )(pa3, at3, pc3, tc3)

    ce2d = ce_b3.reshape(ep // LANES, LANES)
    idx2d = jnp.pad(bond_aggregation_index, (0, ep - e),
                    constant_values=n).reshape(ep // LANES, LANES)
    mse2d = mse3.reshape(na // LANES, LANES)
    cea2d = ce_a3.reshape(na // LANES, LANES)
    bat2d = jnp.pad(batch, (0, na - n), constant_values=b).reshape(na // LANES, LANES)
    zeros_hbm = jnp.zeros((na,), f32)
    w_pad = jnp.pad(weights, (0, bp - b))

    mesh = plsc.VectorSubcoreMesh(core_axis_name="c", subcore_axis_name="s")

    sc1 = pl.kernel(
        functools.partial(_sc_scatter, nt_e, nt_a, na, bp),
        mesh=mesh,
        out_type=[
            jax.ShapeDtypeStruct((2 * na,), f32),
            jax.ShapeDtypeStruct((2 * na,), f32),
            jax.ShapeDtypeStruct((2 * bp,), f32),
            jax.ShapeDtypeStruct((2 * bp,), f32),
            jax.ShapeDtypeStruct((2 * bp,), f32),
        ],
        scratch_types=[
            pltpu.VMEM_SHARED((na,), f32),
            pltpu.VMEM_SHARED((na,), f32),
            pltpu.VMEM_SHARED((bp,), f32),
            pltpu.VMEM_SHARED((bp,), f32),
            pltpu.VMEM_SHARED((bp,), f32),
            pltpu.VMEM((nt_e, LANES), f32),
            pltpu.VMEM((nt_e, LANES), i32),
            pltpu.VMEM((nt_a, LANES), f32),
            pltpu.VMEM((nt_a, LANES), f32),
            pltpu.VMEM((nt_a, LANES), i32),
            pltpu.VMEM((LANES,), f32),
        ],
    )
    s2, c2, n2, ms2, cs2 = sc1(ce2d, idx2d, mse2d, cea2d, bat2d, zeros_hbm)

    rows = na // LANES // NTILES
    sc2 = pl.kernel(
        functools.partial(_sc_final, rows, na, bp),
        mesh=mesh,
        out_type=[
            jax.ShapeDtypeStruct((2 * SCL,), f32),
            jax.ShapeDtypeStruct((SCL,), f32),
            jax.ShapeDtypeStruct((SCL,), f32),
            jax.ShapeDtypeStruct((SCL,), f32),
        ],
        scratch_types=[
            pltpu.VMEM_SHARED((bp,), f32),
            pltpu.VMEM((rows * LANES,), f32),
            pltpu.VMEM((rows * LANES,), f32),
            pltpu.VMEM((rows * LANES,), f32),
            pltpu.VMEM((rows * LANES,), f32),
            pltpu.VMEM((rows, LANES), i32),
            pltpu.VMEM((rows, LANES), f32),
            pltpu.VMEM((bp,), f32),
            pltpu.VMEM((bp,), f32),
            pltpu.VMEM((bp,), f32),
            pltpu.VMEM((bp,), f32),
            pltpu.VMEM((bp,), f32),
            pltpu.VMEM((bp,), f32),
            pltpu.VMEM((SCL,), f32),
        ],
    )
    bsum, coords_v, atoms_v, w_v = sc2(
        s2, c2, n2, ms2, cs2, w_pad, bat2d, zeros_hbm)

    coords_loss = jnp.sum(coords_v)
    atoms_loss = jnp.sum(atoms_v)
    charges_loss = atoms_loss * jnp.sum(w_v)
    bonds_loss = jnp.sum(bsum)
    return (coords_loss, atoms_loss, charges_loss, bonds_loss)


# speedup vs baseline: 8.7135x; 1.0605x over previous
import functools

import jax
import jax.numpy as jnp
from jax import lax
from jax.experimental import pallas as pl
from jax.experimental.pallas import tpu as pltpu
from jax.experimental.pallas import tpu_sc as plsc

LANES = 128
NTILES = 32
SCL = 16


def _ceil_to(x, m):
    return (x + m - 1) // m * m


def _ce_kernel(x_ref, t_ref, o_ref):
    x = x_ref[0]
    c = x.shape[0]
    m = jnp.max(x, axis=0, keepdims=True)
    e = jnp.exp(x - m)
    s = jnp.sum(e, axis=0, keepdims=True)
    lse = jnp.log(s) + m
    lbl = t_ref[0]
    iota = lax.broadcasted_iota(jnp.int32, (c, x.shape[1]), 0)
    picked = jnp.sum(jnp.where(iota == lbl, x, 0.0), axis=0, keepdims=True)
    o_ref[0] = lse - picked


def _atom_kernel(a_ref, t_ref, pc_ref, tc_ref, ce_ref, mse_ref):
    _ce_kernel(a_ref, t_ref, ce_ref)
    d = pc_ref[0] - tc_ref[0]
    mse_ref[0] = jnp.sum(d * d, axis=0, keepdims=True) * (1.0 / 3.0)


def _sc_scatter(nt_e, nt_a, na, bp,
                ce2d, idx2d, mse2d, cea2d, bat2d, zeros_hbm,
                s_out, c_out, n_out, ms_out, cs_out,
                s_sp, c_sp, n_sp, ms_sp, cs_sp,
                cev, idxv, msev, ceav, batv, ones_v, sem):
    ci = lax.axis_index("c")
    si = lax.axis_index("s")
    wid = ci * 16 + si

    @pl.when(si == 0)
    def _init():
        pltpu.sync_copy(zeros_hbm.at[pl.ds(0, na)], s_sp)
        pltpu.sync_copy(zeros_hbm.at[pl.ds(0, na)], c_sp)
        pltpu.sync_copy(zeros_hbm.at[pl.ds(0, bp)], n_sp)
        pltpu.sync_copy(zeros_hbm.at[pl.ds(0, bp)], ms_sp)
        pltpu.sync_copy(zeros_hbm.at[pl.ds(0, bp)], cs_sp)

    for k in range(LANES // SCL):
        ones_v[pl.ds(k * SCL, SCL)] = jnp.full((SCL,), 1.0, jnp.float32)

    pltpu.sync_copy(ce2d.at[pl.ds(wid * nt_e, nt_e)], cev)
    pltpu.sync_copy(idx2d.at[pl.ds(wid * nt_e, nt_e)], idxv)
    pltpu.sync_copy(mse2d.at[pl.ds(wid * nt_a, nt_a)], msev)
    pltpu.sync_copy(cea2d.at[pl.ds(wid * nt_a, nt_a)], ceav)
    pltpu.sync_copy(bat2d.at[pl.ds(wid * nt_a, nt_a)], batv)

    plsc.subcore_barrier()

    kch = 1
    for cand in (28, 24, 16, 8, 4, 2):
        if nt_e % cand == 0:
            kch = cand
            break

    def _edge_chunk(i, carry):
        descs = []
        for j in range(kch):
            r = i * kch + j
            descs.append(pltpu.async_copy(
                cev.at[r], s_sp.at[idxv.at[r]], sem, add=True))
            descs.append(pltpu.async_copy(
                ones_v, c_sp.at[idxv.at[r]], sem, add=True))
        for dsc in descs:
            dsc.wait()
        return carry

    lax.fori_loop(0, nt_e // kch, _edge_chunk, 0)

    adescs = []
    for r in range(nt_a):
        adescs.append(pltpu.async_copy(
            msev.at[r], ms_sp.at[batv.at[r]], sem, add=True))
        adescs.append(pltpu.async_copy(
            ceav.at[r], cs_sp.at[batv.at[r]], sem, add=True))
        adescs.append(pltpu.async_copy(
            ones_v, n_sp.at[batv.at[r]], sem, add=True))
    for dsc in adescs:
        dsc.wait()

    plsc.subcore_barrier()

    @pl.when(si == 0)
    def _emit():
        pltpu.sync_copy(s_sp, s_out.at[pl.ds(ci * na, na)])
        pltpu.sync_copy(c_sp, c_out.at[pl.ds(ci * na, na)])
        pltpu.sync_copy(n_sp, n_out.at[pl.ds(ci * bp, bp)])
        pltpu.sync_copy(ms_sp, ms_out.at[pl.ds(ci * bp, bp)])
        pltpu.sync_copy(cs_sp, cs_out.at[pl.ds(ci * bp, bp)])


def _sc_final(rows, na, bp,
              s2, c2, n2, ms2, cs2, w_pad, bat2d, zeros_hbm,
              bonds_out, coords_out, atoms_out, charges_out,
              tb_sp,
              s0v, s1v, c0v, c1v, batv, tv, wv, n0v, n1v, msv, csv,
              wdivn_v, accv):
    ci = lax.axis_index("c")
    si = lax.axis_index("s")
    wid = ci * 16 + si
    nb_v = bp // SCL
    ch = rows * LANES

    @pl.when(si == 0)
    def _init():
        pltpu.sync_copy(zeros_hbm.at[pl.ds(0, bp)], tb_sp)

    pltpu.sync_copy(w_pad, wv)
    pltpu.sync_copy(n2.at[pl.ds(0, bp)], n0v)
    pltpu.sync_copy(n2.at[pl.ds(bp, bp)], n1v)
    for k in range(nb_v):
        d = pl.ds(k * SCL, SCL)
        nv = n0v[d] + n1v[d]
        wdivn_v[d] = wv[d] / jnp.maximum(nv, 1.0)

    base = wid * ch
    pltpu.sync_copy(s2.at[pl.ds(base, ch)], s0v)
    pltpu.sync_copy(s2.at[pl.ds(na + base, ch)], s1v)
    pltpu.sync_copy(c2.at[pl.ds(base, ch)], c0v)
    pltpu.sync_copy(c2.at[pl.ds(na + base, ch)], c1v)
    pltpu.sync_copy(bat2d.at[pl.ds(wid * rows, rows)], batv)

    for r in range(rows):
        for j in range(LANES // SCL):
            d = pl.ds(r * LANES + j * SCL, SCL)
            sv = s0v[d] + s1v[d]
            cv = c0v[d] + c1v[d]
            tv[r, pl.ds(j * SCL, SCL)] = 0.5 * sv / jnp.maximum(cv, 1.0)

    plsc.subcore_barrier()
    for r in range(rows):
        pltpu.sync_copy(tv.at[r], tb_sp.at[batv.at[r]], add=True)
    plsc.subcore_barrier()

    @pl.when(si == 0)
    def _emit_bonds():
        pltpu.sync_copy(tb_sp, msv)
        bacc = jnp.zeros((SCL,), jnp.float32)
        for k in range(nb_v):
            d = pl.ds(k * SCL, SCL)
            bacc = bacc + msv[d] * wdivn_v[d]
        accv[...] = bacc
        pltpu.sync_copy(accv, bonds_out.at[pl.ds(ci * SCL, SCL)])

    @pl.when((si == 0) & (ci == 0))
    def _scalars():
        pltpu.sync_copy(ms2.at[pl.ds(0, bp)], msv)
        pltpu.sync_copy(cs2.at[pl.ds(0, bp)], csv)
        cacc = jnp.zeros((SCL,), jnp.float32)
        aacc = jnp.zeros((SCL,), jnp.float32)
        wacc = jnp.zeros((SCL,), jnp.float32)
        for k in range(nb_v):
            d = pl.ds(k * SCL, SCL)
            cacc = cacc + msv[d] * wdivn_v[d]
            aacc = aacc + csv[d] * wdivn_v[d]
            wacc = wacc + wv[d]
        pltpu.sync_copy(ms2.at[pl.ds(bp, bp)], msv)
        pltpu.sync_copy(cs2.at[pl.ds(bp, bp)], csv)
        for k in range(nb_v):
            d = pl.ds(k * SCL, SCL)
            cacc = cacc + msv[d] * wdivn_v[d]
            aacc = aacc + csv[d] * wdivn_v[d]
        accv[...] = cacc
        pltpu.sync_copy(accv, coords_out)
        accv[...] = aacc
        pltpu.sync_copy(accv, atoms_out)
        accv[...] = wacc
        pltpu.sync_copy(accv, charges_out)


def kernel(pred_coords, true_coords, pred_atoms, atoms_target, pred_charges,
           charges_target, pred_bonds, bonds_target, batch,
           bond_aggregation_index, weights):
    n = pred_coords.shape[0]
    e = pred_bonds.shape[0]
    b = weights.shape[0]
    a_cls = pred_atoms.shape[1]
    bond_cls = pred_bonds.shape[1]

    nt_e = _ceil_to(-(-e // (NTILES * LANES)), 8)
    ep = nt_e * NTILES * LANES
    nt_a = _ceil_to(-(-n // (NTILES * LANES)), 8)
    na = nt_a * NTILES * LANES
    bp = _ceil_to(b + 1, LANES)

    f32 = jnp.float32
    i32 = jnp.int32

    KE = NTILES * LANES
    ge = ep // KE
    pb = jnp.pad(pred_bonds, ((0, ep - e), (0, 0)))
    pb3 = jnp.transpose(pb.reshape(ge, KE, bond_cls), (0, 2, 1))
    bt3 = jnp.pad(bonds_target, (0, ep - e)).reshape(ge, 1, KE)

    ga = na // KE
    pa = jnp.pad(pred_atoms, ((0, na - n), (0, 0)))
    pa3 = jnp.transpose(pa.reshape(ga, KE, a_cls), (0, 2, 1))
    at3 = jnp.pad(atoms_target, (0, na - n)).reshape(ga, 1, KE)
    pc3 = jnp.transpose(jnp.pad(pred_coords, ((0, na - n), (0, 0))).reshape(ga, KE, 3), (0, 2, 1))
    tc3 = jnp.transpose(jnp.pad(true_coords, ((0, na - n), (0, 0))).reshape(ga, KE, 3), (0, 2, 1))

    ce_b3 = pl.pallas_call(
        _ce_kernel,
        grid=(ge,),
        in_specs=[
            pl.BlockSpec((1, bond_cls, KE), lambda i: (i, 0, 0)),
            pl.BlockSpec((1, 1, KE), lambda i: (i, 0, 0)),
        ],
        out_specs=pl.BlockSpec((1, 1, KE), lambda i: (i, 0, 0)),
        out_shape=jax.ShapeDtypeStruct((ge, 1, KE), f32),
    )(pb3, bt3)

    ce_a3, mse3 = pl.pallas_call(
        _atom_kernel,
        grid=(ga,),
        in_specs=[
            pl.BlockSpec((1, a_cls, KE), lambda i: (i, 0, 0)),
            pl.BlockSpec((1, 1, KE), lambda i: (i, 0, 0)),
            pl.BlockSpec((1, 3, KE), lambda i: (i, 0, 0)),
            pl.BlockSpec((1, 3, KE), lambda i: (i, 0, 0)),
        ],
        out_specs=[
            pl.BlockSpec((1, 1, KE), lambda i: (i, 0, 0)),
            pl.BlockSpec((1, 1, KE), lambda i: (i, 0, 0)),
        ],
        out_shape=[
            jax.ShapeDtypeStruct((ga, 1, KE), f32),
            jax.ShapeDtypeStruct((ga, 1, KE), f32),
        ],
    )(pa3, at3, pc3, tc3)

    ce2d = ce_b3.reshape(ep // LANES, LANES)
    idx2d = jnp.pad(bond_aggregation_index, (0, ep - e),
                    constant_values=n).reshape(ep // LANES, LANES)
    mse2d = mse3.reshape(na // LANES, LANES)
    cea2d = ce_a3.reshape(na // LANES, LANES)
    bat2d = jnp.pad(batch, (0, na - n), constant_values=b).reshape(na // LANES, LANES)
    zeros_hbm = jnp.zeros((na,), f32)
    w_pad = jnp.pad(weights, (0, bp - b))

    mesh = plsc.VectorSubcoreMesh(core_axis_name="c", subcore_axis_name="s")

    sc1 = pl.kernel(
        functools.partial(_sc_scatter, nt_e, nt_a, na, bp),
        mesh=mesh,
        out_type=[
            jax.ShapeDtypeStruct((2 * na,), f32),
            jax.ShapeDtypeStruct((2 * na,), f32),
            jax.ShapeDtypeStruct((2 * bp,), f32),
            jax.ShapeDtypeStruct((2 * bp,), f32),
            jax.ShapeDtypeStruct((2 * bp,), f32),
        ],
        scratch_types=[
            pltpu.VMEM_SHARED((na,), f32),
            pltpu.VMEM_SHARED((na,), f32),
            pltpu.VMEM_SHARED((bp,), f32),
            pltpu.VMEM_SHARED((bp,), f32),
            pltpu.VMEM_SHARED((bp,), f32),
            pltpu.VMEM((nt_e, LANES), f32),
            pltpu.VMEM((nt_e, LANES), i32),
            pltpu.VMEM((nt_a, LANES), f32),
            pltpu.VMEM((nt_a, LANES), f32),
            pltpu.VMEM((nt_a, LANES), i32),
            pltpu.VMEM((LANES,), f32),
            pltpu.SemaphoreType.DMA,
        ],
    )
    s2, c2, n2, ms2, cs2 = sc1(ce2d, idx2d, mse2d, cea2d, bat2d, zeros_hbm)

    rows = na // LANES // NTILES
    sc2 = pl.kernel(
        functools.partial(_sc_final, rows, na, bp),
        mesh=mesh,
        out_type=[
            jax.ShapeDtypeStruct((2 * SCL,), f32),
            jax.ShapeDtypeStruct((SCL,), f32),
            jax.ShapeDtypeStruct((SCL,), f32),
            jax.ShapeDtypeStruct((SCL,), f32),
        ],
        scratch_types=[
            pltpu.VMEM_SHARED((bp,), f32),
            pltpu.VMEM((rows * LANES,), f32),
            pltpu.VMEM((rows * LANES,), f32),
            pltpu.VMEM((rows * LANES,), f32),
            pltpu.VMEM((rows * LANES,), f32),
            pltpu.VMEM((rows, LANES), i32),
            pltpu.VMEM((rows, LANES), f32),
            pltpu.VMEM((bp,), f32),
            pltpu.VMEM((bp,), f32),
            pltpu.VMEM((bp,), f32),
            pltpu.VMEM((bp,), f32),
            pltpu.VMEM((bp,), f32),
            pltpu.VMEM((bp,), f32),
            pltpu.VMEM((SCL,), f32),
        ],
    )
    bsum, coords_v, atoms_v, w_v = sc2(
        s2, c2, n2, ms2, cs2, w_pad, bat2d, zeros_hbm)

    coords_loss = jnp.sum(coords_v)
    atoms_loss = jnp.sum(atoms_v)
    charges_loss = atoms_loss * jnp.sum(w_v)
    bonds_loss = jnp.sum(bsum)
    return (coords_loss, atoms_loss, charges_loss, bonds_loss)
